# baseline (speedup 1.0000x reference)
import functools

import jax
import jax.numpy as jnp
from jax import lax
from jax.experimental import pallas as pl
from jax.experimental.pallas import tpu as pltpu
from jax.experimental.pallas import tpu_sc as plsc

B = 8
N = 512
DS = 256
VOCAB = 32
R2 = 36.0
NLAYERS = 4

NW = 32
RPW = (B * N) // NW
SLABS_PER_B = N // RPW
ECAP = RPW * N
TILE = 128
LANES = 16
CH = 16


def _adjacency_xla(tokens, coords):
    coord = coords.reshape(-1, 3)
    valid = (tokens != 0).reshape(-1)
    batch = jnp.repeat(jnp.arange(B), N)
    sq = jnp.sum(coord * coord, axis=-1)
    d2 = sq[:, None] + sq[None, :] - 2.0 * (coord @ coord.T)
    d2 = jnp.maximum(d2, 0.0)
    same = batch[:, None] == batch[None, :]
    m = coord.shape[0]
    adj = (d2 < R2) & same & (~jnp.eye(m, dtype=bool))
    adj = adj & valid[:, None] & valid[None, :]
    adjd = jnp.stack([adj[i * N:(i + 1) * N, i * N:(i + 1) * N]
                      for i in range(B)])
    return adjd.astype(jnp.int32)


def _sc_compact_body(adj_hbm, edges_hbm, counts_hbm, rows_v, edges_v, cnt_v):
    cid = lax.axis_index("c")
    sid = lax.axis_index("s")
    w = cid * 16 + sid
    rb0 = (w % SLABS_PER_B) * RPW
    iota16 = lax.iota(jnp.int32, LANES)
    vpc = CH * N // LANES
    rpc = CH * N // 128

    def chunk_loop(ch, off):
        pltpu.sync_copy(
            adj_hbm.at[pl.ds(pl.multiple_of((w * RPW + ch * CH) * N // 128, 8), rpc)],
            rows_v)
        pkb = (rb0 + ch * CH) * N

        def vec_loop(p, off):
            flatv = iota16 + p * LANES
            vec = plsc.load_gather(rows_v, [flatv >> 7, flatv & 127])
            mask = vec != 0
            mvec = mask.astype(jnp.int32)
            cum = jnp.cumsum(mvec)
            pk = flatv + pkb
            pos = jnp.where(mask, off + cum - 1, ECAP + iota16)
            plsc.store_scatter(edges_v, [pos >> 7, pos & 127], pk)
            return off + jnp.sum(mvec)

        return lax.fori_loop(0, vpc, vec_loop, off)

    total = lax.fori_loop(0, RPW // CH, chunk_loop, jnp.int32(0))
    pltpu.sync_copy(edges_v.at[pl.ds(0, ECAP // 128)], edges_hbm.at[w])
    cnt_v[0] = jnp.zeros((LANES,), jnp.int32) + total
    pltpu.sync_copy(cnt_v, counts_hbm.at[w])


def _compact_edges_sc(adj):
    f = pl.kernel(
        _sc_compact_body,
        mesh=plsc.VectorSubcoreMesh(core_axis_name="c", subcore_axis_name="s"),
        out_type=[
            jax.ShapeDtypeStruct((NW, ECAP // 128, 128), jnp.int32),
            jax.ShapeDtypeStruct((NW, 8, LANES), jnp.int32),
        ],
        scratch_types=[
            pltpu.VMEM((CH * N // 128, 128), jnp.int32),
            pltpu.VMEM((ECAP // 128 + 1, 128), jnp.int32),
            pltpu.VMEM((8, LANES), jnp.int32),
        ],
        compiler_params=pltpu.CompilerParams(needs_layout_passes=False),
    )
    edges, counts = f(adj.reshape(B * N * N // 128, 128))
    return edges.reshape(B, SLABS_PER_B, ECAP), counts[:, 0, 0]


def _silu(x):
    return x * jax.nn.sigmoid(x)


def _dot(x, y):
    return lax.dot_general(x, y, (((1,), (0,)), ((), ())),
                           preferred_element_type=jnp.float32)


def _dot_t(x, y):
    return lax.dot_general(x, y, (((0,), (0,)), ((), ())),
                           preferred_element_type=jnp.float32)


def _mp_body(counts_ref, tokc_ref, deg_ref, edges_ref, emb_ref,
             at_ref, bt_ref, b1_ref, w2t_ref, b2_ref,
             u1t_ref, bu1_ref, u2t_ref, bu2_ref, out_ref):
    b = pl.program_id(0)
    tok_c = tokc_ref[0]
    deg = deg_ref[0]
    onehot = (tok_c == lax.broadcasted_iota(jnp.int32, (N, VOCAB), 1))
    s = _dot(onehot.astype(jnp.float32), emb_ref[...])
    lane = lax.broadcasted_iota(jnp.int32, (1, TILE), 1)
    row_iota = lax.broadcasted_iota(jnp.int32, (RPW, TILE), 0)
    col_iota = lax.broadcasted_iota(jnp.int32, (N, TILE), 0)

    for l in range(NLAYERS):
        a = _dot(s, at_ref[l]) + b1_ref[l]
        bm = _dot(s, bt_ref[l])
        aggs = []
        for reg in range(SLABS_PER_B):
            cnt = counts_ref[b * SLABS_PER_B + reg]
            a_slab = a[reg * RPW:(reg + 1) * RPW, :]

            def tile_body(t, hslab, _reg=reg, _cnt=cnt, _a_slab=a_slab, _bm=bm):
                base = t * TILE
                e = edges_ref[0, _reg, pl.ds(base, TILE)].reshape(1, TILE)
                r = e >> 9
                c = e & (N - 1)
                ev = (base + lane) < _cnt
                grt = (((row_iota + _reg * RPW) == r) & ev).astype(jnp.float32)
                gct = ((col_iota == c) & ev).astype(jnp.float32)
                ar = _dot_t(grt, _a_slab)
                bc = _dot_t(gct, _bm)
                h = _silu(ar + bc)
                return hslab + _dot(grt, h)

            n_tiles = (cnt + (TILE - 1)) // TILE
            hslab = lax.fori_loop(0, n_tiles, tile_body,
                                  jnp.zeros((RPW, 2 * DS), jnp.float32))
            aggs.append(_dot(hslab, w2t_ref[l]))
        agg = jnp.concatenate(aggs, axis=0) + deg * b2_ref[l]
        u = _silu(_dot(agg, u1t_ref[l]) + bu1_ref[l])
        s = s + _dot(u, u2t_ref[l]) + bu2_ref[l]

    out_ref[0] = jnp.where(tok_c != 0, s, 0.0)


def _message_passing(counts, tok_c, deg, edges, emb, stacked):
    at, bt, b1, w2t, b2, u1t, bu1, u2t, bu2 = stacked
    full = lambda shape: pl.BlockSpec(shape, lambda b, *_: tuple(0 for _ in shape))
    grid_spec = pltpu.PrefetchScalarGridSpec(
        num_scalar_prefetch=1,
        grid=(B,),
        in_specs=[
            pl.BlockSpec((1, N, 1), lambda b, *_: (b, 0, 0)),
            pl.BlockSpec((1, N, 1), lambda b, *_: (b, 0, 0)),
            pl.BlockSpec((1, SLABS_PER_B, ECAP), lambda b, *_: (b, 0, 0)),
            full((VOCAB, DS)),
            full((NLAYERS, DS, 2 * DS)),
            full((NLAYERS, DS, 2 * DS)),
            full((NLAYERS, 1, 2 * DS)),
            full((NLAYERS, 2 * DS, DS)),
            full((NLAYERS, 1, DS)),
            full((NLAYERS, DS, DS)),
            full((NLAYERS, 1, DS)),
            full((NLAYERS, DS, DS)),
            full((NLAYERS, 1, DS)),
        ],
        out_specs=pl.BlockSpec((1, N, DS), lambda b, *_: (b, 0, 0)),
    )
    return pl.pallas_call(
        _mp_body,
        grid_spec=grid_spec,
        out_shape=jax.ShapeDtypeStruct((B, N, DS), jnp.float32),
    )(counts, tok_c, deg, edges, emb, at, bt, b1, w2t, b2, u1t, bu1, u2t, bu2)


def _stack_weights(params):
    ls = params["layers"]
    at = jnp.stack([lp["edge_s1"]["W"][:, :DS].T for lp in ls])
    bt = jnp.stack([lp["edge_s1"]["W"][:, DS:].T for lp in ls])
    b1 = jnp.stack([lp["edge_s1"]["b"].reshape(1, 2 * DS) for lp in ls])
    w2t = jnp.stack([lp["edge_s2"]["W"].T for lp in ls])
    b2 = jnp.stack([lp["edge_s2"]["b"].reshape(1, DS) for lp in ls])
    u1t = jnp.stack([lp["upd_s1"]["W"].T for lp in ls])
    bu1 = jnp.stack([lp["upd_s1"]["b"].reshape(1, DS) for lp in ls])
    u2t = jnp.stack([lp["upd_s2"]["W"].T for lp in ls])
    bu2 = jnp.stack([lp["upd_s2"]["b"].reshape(1, DS) for lp in ls])
    return at, bt, b1, w2t, b2, u1t, bu1, u2t, bu2


@jax.jit
def kernel(src_tokens, padded_coordinates, src_distance, src_edge_type, params):
    del src_distance, src_edge_type
    tokens = src_tokens.astype(jnp.int32)
    padding_mask = src_tokens == 0
    adj = _adjacency_xla(tokens, padded_coordinates)
    deg = jnp.sum(adj.astype(jnp.float32), axis=2).reshape(B, N, 1)
    edges, counts = _compact_edges_sc(adj)
    tok_c = tokens.reshape(B, N, 1)
    stacked = _stack_weights(params)
    out = _message_passing(counts, tok_c, deg, edges, params["emb"], stacked)
    return out, padding_mask


# speedup vs baseline: 1.3413x; 1.3413x over previous
import functools

import jax
import jax.numpy as jnp
from jax import lax
from jax.experimental import pallas as pl
from jax.experimental.pallas import tpu as pltpu
from jax.experimental.pallas import tpu_sc as plsc

B = 8
N = 512
DS = 256
VOCAB = 32
R2 = 36.0
NLAYERS = 4

NW = 32
RPW = (B * N) // NW
SLABS_PER_B = N // RPW
ECAP = RPW * N
TILE = 512
LANES = 16
CH = 16


def _adjacency_xla(tokens, coords):
    coord = coords.reshape(-1, 3)
    valid = (tokens != 0).reshape(-1)
    batch = jnp.repeat(jnp.arange(B), N)
    sq = jnp.sum(coord * coord, axis=-1)
    d2 = sq[:, None] + sq[None, :] - 2.0 * (coord @ coord.T)
    d2 = jnp.maximum(d2, 0.0)
    same = batch[:, None] == batch[None, :]
    m = coord.shape[0]
    adj = (d2 < R2) & same & (~jnp.eye(m, dtype=bool))
    adj = adj & valid[:, None] & valid[None, :]
    adjd = jnp.stack([adj[i * N:(i + 1) * N, i * N:(i + 1) * N]
                      for i in range(B)])
    return adjd.astype(jnp.int32)


def _sc_compact_body(adj_hbm, edges_hbm, counts_hbm, rows_v, edges_v, cnt_v):
    cid = lax.axis_index("c")
    sid = lax.axis_index("s")
    w = cid * 16 + sid
    rb0 = (w % SLABS_PER_B) * RPW
    iota16 = lax.iota(jnp.int32, LANES)
    vpc = CH * N // LANES
    rpc = CH * N // 128

    def chunk_loop(ch, off):
        pltpu.sync_copy(
            adj_hbm.at[pl.ds(pl.multiple_of((w * RPW + ch * CH) * N // 128, 8), rpc)],
            rows_v)
        pkb = (rb0 + ch * CH) * N

        def vec_loop(p, off):
            flatv = iota16 + p * LANES
            vec = plsc.load_gather(rows_v, [flatv >> 7, flatv & 127])
            mask = vec != 0
            mvec = mask.astype(jnp.int32)
            cum = jnp.cumsum(mvec)
            pk = flatv + pkb
            pos = jnp.where(mask, off + cum - 1, ECAP + iota16)
            plsc.store_scatter(edges_v, [pos >> 7, pos & 127], pk)
            return off + jnp.sum(mvec)

        return lax.fori_loop(0, vpc, vec_loop, off)

    total = lax.fori_loop(0, RPW // CH, chunk_loop, jnp.int32(0))
    pltpu.sync_copy(edges_v.at[pl.ds(0, ECAP // 128)], edges_hbm.at[w])
    cnt_v[0] = jnp.zeros((LANES,), jnp.int32) + total
    pltpu.sync_copy(cnt_v, counts_hbm.at[w])


def _compact_edges_sc(adj):
    f = pl.kernel(
        _sc_compact_body,
        mesh=plsc.VectorSubcoreMesh(core_axis_name="c", subcore_axis_name="s"),
        out_type=[
            jax.ShapeDtypeStruct((NW, ECAP // 128, 128), jnp.int32),
            jax.ShapeDtypeStruct((NW, 8, LANES), jnp.int32),
        ],
        scratch_types=[
            pltpu.VMEM((CH * N // 128, 128), jnp.int32),
            pltpu.VMEM((ECAP // 128 + 1, 128), jnp.int32),
            pltpu.VMEM((8, LANES), jnp.int32),
        ],
        compiler_params=pltpu.CompilerParams(needs_layout_passes=False),
    )
    edges, counts = f(adj.reshape(B * N * N // 128, 128))
    return edges.reshape(B, SLABS_PER_B, ECAP), counts[:, 0, 0]


def _silu(x):
    return x * jax.nn.sigmoid(x)


def _dot(x, y):
    return lax.dot_general(x, y, (((1,), (0,)), ((), ())),
                           preferred_element_type=jnp.float32)


def _dot_t(x, y):
    return lax.dot_general(x, y, (((0,), (0,)), ((), ())),
                           preferred_element_type=jnp.float32)


def _mp_body(counts_ref, tokc_ref, deg_ref, edges_ref, emb_ref,
             at_ref, bt_ref, b1_ref, w2t_ref, b2_ref,
             u1t_ref, bu1_ref, u2t_ref, bu2_ref, out_ref):
    b = pl.program_id(0)
    tok_c = tokc_ref[0]
    deg = deg_ref[0]
    onehot = (tok_c == lax.broadcasted_iota(jnp.int32, (N, VOCAB), 1))
    s = _dot(onehot.astype(jnp.float32), emb_ref[...])
    lane = lax.broadcasted_iota(jnp.int32, (1, TILE), 1)
    row_iota = lax.broadcasted_iota(jnp.int32, (RPW, TILE), 0)
    col_iota = lax.broadcasted_iota(jnp.int32, (N, TILE), 0)

    for l in range(NLAYERS):
        a = _dot(s, at_ref[l]) + b1_ref[l]
        bm = _dot(s, bt_ref[l])
        aggs = []
        for reg in range(SLABS_PER_B):
            cnt = counts_ref[b * SLABS_PER_B + reg]
            a_slab = a[reg * RPW:(reg + 1) * RPW, :]

            def tile_body(t, hslab, _reg=reg, _cnt=cnt, _a_slab=a_slab, _bm=bm):
                base = t * TILE
                e = edges_ref[0, _reg, pl.ds(base, TILE)].reshape(1, TILE)
                r = e >> 9
                c = e & (N - 1)
                ev = (base + lane) < _cnt
                grt = (((row_iota + _reg * RPW) == r) & ev).astype(jnp.float32)
                gct = ((col_iota == c) & ev).astype(jnp.float32)
                ar = _dot_t(grt, _a_slab)
                bc = _dot_t(gct, _bm)
                h = _silu(ar + bc)
                return hslab + _dot(grt, h)

            n_tiles = (cnt + (TILE - 1)) // TILE
            hslab = lax.fori_loop(0, n_tiles, tile_body,
                                  jnp.zeros((RPW, 2 * DS), jnp.float32))
            aggs.append(_dot(hslab, w2t_ref[l]))
        agg = jnp.concatenate(aggs, axis=0) + deg * b2_ref[l]
        u = _silu(_dot(agg, u1t_ref[l]) + bu1_ref[l])
        s = s + _dot(u, u2t_ref[l]) + bu2_ref[l]

    out_ref[0] = jnp.where(tok_c != 0, s, 0.0)


def _message_passing(counts, tok_c, deg, edges, emb, stacked):
    at, bt, b1, w2t, b2, u1t, bu1, u2t, bu2 = stacked
    full = lambda shape: pl.BlockSpec(shape, lambda b, *_: tuple(0 for _ in shape))
    grid_spec = pltpu.PrefetchScalarGridSpec(
        num_scalar_prefetch=1,
        grid=(B,),
        in_specs=[
            pl.BlockSpec((1, N, 1), lambda b, *_: (b, 0, 0)),
            pl.BlockSpec((1, N, 1), lambda b, *_: (b, 0, 0)),
            pl.BlockSpec((1, SLABS_PER_B, ECAP), lambda b, *_: (b, 0, 0)),
            full((VOCAB, DS)),
            full((NLAYERS, DS, 2 * DS)),
            full((NLAYERS, DS, 2 * DS)),
            full((NLAYERS, 1, 2 * DS)),
            full((NLAYERS, 2 * DS, DS)),
            full((NLAYERS, 1, DS)),
            full((NLAYERS, DS, DS)),
            full((NLAYERS, 1, DS)),
            full((NLAYERS, DS, DS)),
            full((NLAYERS, 1, DS)),
        ],
        out_specs=pl.BlockSpec((1, N, DS), lambda b, *_: (b, 0, 0)),
    )
    return pl.pallas_call(
        _mp_body,
        grid_spec=grid_spec,
        out_shape=jax.ShapeDtypeStruct((B, N, DS), jnp.float32),
    )(counts, tok_c, deg, edges, emb, at, bt, b1, w2t, b2, u1t, bu1, u2t, bu2)


def _stack_weights(params):
    ls = params["layers"]
    at = jnp.stack([lp["edge_s1"]["W"][:, :DS].T for lp in ls])
    bt = jnp.stack([lp["edge_s1"]["W"][:, DS:].T for lp in ls])
    b1 = jnp.stack([lp["edge_s1"]["b"].reshape(1, 2 * DS) for lp in ls])
    w2t = jnp.stack([lp["edge_s2"]["W"].T for lp in ls])
    b2 = jnp.stack([lp["edge_s2"]["b"].reshape(1, DS) for lp in ls])
    u1t = jnp.stack([lp["upd_s1"]["W"].T for lp in ls])
    bu1 = jnp.stack([lp["upd_s1"]["b"].reshape(1, DS) for lp in ls])
    u2t = jnp.stack([lp["upd_s2"]["W"].T for lp in ls])
    bu2 = jnp.stack([lp["upd_s2"]["b"].reshape(1, DS) for lp in ls])
    return at, bt, b1, w2t, b2, u1t, bu1, u2t, bu2


@jax.jit
def kernel(src_tokens, padded_coordinates, src_distance, src_edge_type, params):
    del src_distance, src_edge_type
    tokens = src_tokens.astype(jnp.int32)
    padding_mask = src_tokens == 0
    adj = _adjacency_xla(tokens, padded_coordinates)
    deg = jnp.sum(adj.astype(jnp.float32), axis=2).reshape(B, N, 1)
    edges, counts = _compact_edges_sc(adj)
    tok_c = tokens.reshape(B, N, 1)
    stacked = _stack_weights(params)
    out = _message_passing(counts, tok_c, deg, edges, params["emb"], stacked)
    return out, padding_mask


# speedup vs baseline: 1.4474x; 1.0791x over previous
import functools

import jax
import jax.numpy as jnp
from jax import lax
from jax.experimental import pallas as pl
from jax.experimental.pallas import tpu as pltpu
from jax.experimental.pallas import tpu_sc as plsc

B = 8
N = 512
DS = 256
VOCAB = 32
R2 = 36.0
NLAYERS = 4

NW = 32
RPW = (B * N) // NW
SLABS_PER_B = N // RPW
ECAP = RPW * N
TILE = 1024
LANES = 16
CH = 16


def _adjacency_xla(tokens, coords):
    coord = coords.reshape(-1, 3)
    valid = (tokens != 0).reshape(-1)
    batch = jnp.repeat(jnp.arange(B), N)
    sq = jnp.sum(coord * coord, axis=-1)
    d2 = sq[:, None] + sq[None, :] - 2.0 * (coord @ coord.T)
    d2 = jnp.maximum(d2, 0.0)
    same = batch[:, None] == batch[None, :]
    m = coord.shape[0]
    adj = (d2 < R2) & same & (~jnp.eye(m, dtype=bool))
    adj = adj & valid[:, None] & valid[None, :]
    adjd = jnp.stack([adj[i * N:(i + 1) * N, i * N:(i + 1) * N]
                      for i in range(B)])
    return adjd.astype(jnp.int32)


def _sc_compact_body(adj_hbm, edges_hbm, counts_hbm, rows_v, edges_v, cnt_v):
    cid = lax.axis_index("c")
    sid = lax.axis_index("s")
    w = cid * 16 + sid
    rb0 = (w % SLABS_PER_B) * RPW
    iota16 = lax.iota(jnp.int32, LANES)
    vpc = CH * N // LANES
    rpc = CH * N // 128

    def chunk_loop(ch, off):
        pltpu.sync_copy(
            adj_hbm.at[pl.ds(pl.multiple_of((w * RPW + ch * CH) * N // 128, 8), rpc)],
            rows_v)
        pkb = (rb0 + ch * CH) * N

        def vec_loop(p, off):
            flatv = iota16 + p * LANES
            vec = plsc.load_gather(rows_v, [flatv >> 7, flatv & 127])
            mask = vec != 0
            mvec = mask.astype(jnp.int32)
            cum = jnp.cumsum(mvec)
            pk = flatv + pkb
            pos = jnp.where(mask, off + cum - 1, ECAP + iota16)
            plsc.store_scatter(edges_v, [pos >> 7, pos & 127], pk)
            return off + jnp.sum(mvec)

        return lax.fori_loop(0, vpc, vec_loop, off)

    total = lax.fori_loop(0, RPW // CH, chunk_loop, jnp.int32(0))
    pltpu.sync_copy(edges_v.at[pl.ds(0, ECAP // 128)], edges_hbm.at[w])
    cnt_v[0] = jnp.zeros((LANES,), jnp.int32) + total
    pltpu.sync_copy(cnt_v, counts_hbm.at[w])


def _compact_edges_sc(adj):
    f = pl.kernel(
        _sc_compact_body,
        mesh=plsc.VectorSubcoreMesh(core_axis_name="c", subcore_axis_name="s"),
        out_type=[
            jax.ShapeDtypeStruct((NW, ECAP // 128, 128), jnp.int32),
            jax.ShapeDtypeStruct((NW, 8, LANES), jnp.int32),
        ],
        scratch_types=[
            pltpu.VMEM((CH * N // 128, 128), jnp.int32),
            pltpu.VMEM((ECAP // 128 + 1, 128), jnp.int32),
            pltpu.VMEM((8, LANES), jnp.int32),
        ],
        compiler_params=pltpu.CompilerParams(needs_layout_passes=False),
    )
    edges, counts = f(adj.reshape(B * N * N // 128, 128))
    return edges.reshape(B, SLABS_PER_B, ECAP), counts[:, 0, 0]


def _silu(x):
    return x * jax.nn.sigmoid(x)


def _dot(x, y):
    return lax.dot_general(x, y, (((1,), (0,)), ((), ())),
                           preferred_element_type=jnp.float32)


def _dot_t(x, y):
    return lax.dot_general(x, y, (((0,), (0,)), ((), ())),
                           preferred_element_type=jnp.float32)


def _mp_body(counts_ref, tokc_ref, deg_ref, edges_ref, emb_ref,
             at_ref, bt_ref, b1_ref, w2t_ref, b2_ref,
             u1t_ref, bu1_ref, u2t_ref, bu2_ref, out_ref):
    b = pl.program_id(0)
    tok_c = tokc_ref[0]
    deg = deg_ref[0]
    onehot = (tok_c == lax.broadcasted_iota(jnp.int32, (N, VOCAB), 1))
    s = _dot(onehot.astype(jnp.float32), emb_ref[...])
    lane = lax.broadcasted_iota(jnp.int32, (1, TILE), 1)
    row_iota = lax.broadcasted_iota(jnp.int32, (RPW, TILE), 0)
    col_iota = lax.broadcasted_iota(jnp.int32, (N, TILE), 0)

    for l in range(NLAYERS):
        a = _dot(s, at_ref[l]) + b1_ref[l]
        bm = _dot(s, bt_ref[l])
        aggs = []
        for reg in range(SLABS_PER_B):
            cnt = counts_ref[b * SLABS_PER_B + reg]
            a_slab = a[reg * RPW:(reg + 1) * RPW, :]

            def tile_body(t, hslab, _reg=reg, _cnt=cnt, _a_slab=a_slab, _bm=bm):
                base = t * TILE
                e = edges_ref[0, _reg, pl.ds(base, TILE)].reshape(1, TILE)
                r = e >> 9
                c = e & (N - 1)
                ev = (base + lane) < _cnt
                grt = (((row_iota + _reg * RPW) == r) & ev).astype(jnp.float32)
                gct = ((col_iota == c) & ev).astype(jnp.float32)
                ar = _dot_t(grt, _a_slab)
                bc = _dot_t(gct, _bm)
                h = _silu(ar + bc)
                return hslab + _dot(grt, h)

            n_tiles = (cnt + (TILE - 1)) // TILE
            hslab = lax.fori_loop(0, n_tiles, tile_body,
                                  jnp.zeros((RPW, 2 * DS), jnp.float32))
            aggs.append(_dot(hslab, w2t_ref[l]))
        agg = jnp.concatenate(aggs, axis=0) + deg * b2_ref[l]
        u = _silu(_dot(agg, u1t_ref[l]) + bu1_ref[l])
        s = s + _dot(u, u2t_ref[l]) + bu2_ref[l]

    out_ref[0] = jnp.where(tok_c != 0, s, 0.0)


def _message_passing(counts, tok_c, deg, edges, emb, stacked):
    at, bt, b1, w2t, b2, u1t, bu1, u2t, bu2 = stacked
    full = lambda shape: pl.BlockSpec(shape, lambda b, *_: tuple(0 for _ in shape))
    grid_spec = pltpu.PrefetchScalarGridSpec(
        num_scalar_prefetch=1,
        grid=(B,),
        in_specs=[
            pl.BlockSpec((1, N, 1), lambda b, *_: (b, 0, 0)),
            pl.BlockSpec((1, N, 1), lambda b, *_: (b, 0, 0)),
            pl.BlockSpec((1, SLABS_PER_B, ECAP), lambda b, *_: (b, 0, 0)),
            full((VOCAB, DS)),
            full((NLAYERS, DS, 2 * DS)),
            full((NLAYERS, DS, 2 * DS)),
            full((NLAYERS, 1, 2 * DS)),
            full((NLAYERS, 2 * DS, DS)),
            full((NLAYERS, 1, DS)),
            full((NLAYERS, DS, DS)),
            full((NLAYERS, 1, DS)),
            full((NLAYERS, DS, DS)),
            full((NLAYERS, 1, DS)),
        ],
        out_specs=pl.BlockSpec((1, N, DS), lambda b, *_: (b, 0, 0)),
    )
    return pl.pallas_call(
        _mp_body,
        grid_spec=grid_spec,
        out_shape=jax.ShapeDtypeStruct((B, N, DS), jnp.float32),
    )(counts, tok_c, deg, edges, emb, at, bt, b1, w2t, b2, u1t, bu1, u2t, bu2)


def _stack_weights(params):
    ls = params["layers"]
    at = jnp.stack([lp["edge_s1"]["W"][:, :DS].T for lp in ls])
    bt = jnp.stack([lp["edge_s1"]["W"][:, DS:].T for lp in ls])
    b1 = jnp.stack([lp["edge_s1"]["b"].reshape(1, 2 * DS) for lp in ls])
    w2t = jnp.stack([lp["edge_s2"]["W"].T for lp in ls])
    b2 = jnp.stack([lp["edge_s2"]["b"].reshape(1, DS) for lp in ls])
    u1t = jnp.stack([lp["upd_s1"]["W"].T for lp in ls])
    bu1 = jnp.stack([lp["upd_s1"]["b"].reshape(1, DS) for lp in ls])
    u2t = jnp.stack([lp["upd_s2"]["W"].T for lp in ls])
    bu2 = jnp.stack([lp["upd_s2"]["b"].reshape(1, DS) for lp in ls])
    return at, bt, b1, w2t, b2, u1t, bu1, u2t, bu2


@jax.jit
def kernel(src_tokens, padded_coordinates, src_distance, src_edge_type, params):
    del src_distance, src_edge_type
    tokens = src_tokens.astype(jnp.int32)
    padding_mask = src_tokens == 0
    adj = _adjacency_xla(tokens, padded_coordinates)
    deg = jnp.sum(adj.astype(jnp.float32), axis=2).reshape(B, N, 1)
    edges, counts = _compact_edges_sc(adj)
    tok_c = tokens.reshape(B, N, 1)
    stacked = _stack_weights(params)
    out = _message_passing(counts, tok_c, deg, edges, params["emb"], stacked)
    return out, padding_mask
